# baseline (speedup 1.0000x reference)
import functools

import jax
import jax.numpy as jnp
from jax import lax
from jax.experimental import pallas as pl
from jax.experimental.pallas import tpu as pltpu
from jax.experimental.pallas import tpu_sc as plsc

B, N, E, H, V, C = 8, 2048, 32768, 128, 50000, 2
NGS = 2 * B

_info = plsc.get_sparse_core_info()
NC, NS = _info.num_cores, _info.num_subcores
NW = NC * NS


def _gather_rows(table, idx):
    n = idx.shape[0]
    per_w = n // NW
    chunk = 512
    mesh = plsc.VectorSubcoreMesh(core_axis_name="c", subcore_axis_name="s")

    @functools.partial(
        pl.kernel, mesh=mesh,
        out_type=jax.ShapeDtypeStruct((n, H), jnp.float32),
        scratch_types=[
            pltpu.VMEM((chunk,), jnp.int32),
            pltpu.VMEM((chunk, H), jnp.float32),
            pltpu.SemaphoreType.DMA,
        ],
    )
    def k(table_hbm, idx_hbm, out_hbm, idx_v, rows_v, sem):
        wid = lax.axis_index("s") * NC + lax.axis_index("c")
        base = wid * per_w

        def body(i, carry):
            off = base + i * chunk
            pltpu.sync_copy(idx_hbm.at[pl.ds(off, chunk)], idx_v)
            pltpu.async_copy(table_hbm.at[idx_v], rows_v, sem).wait()
            pltpu.sync_copy(rows_v, out_hbm.at[pl.ds(off, chunk)])
            return carry

        lax.fori_loop(0, per_w // chunk, body, 0)

    return k(table, idx)


_EK = 32

_GDN = lax.GatherDimensionNumbers(
    offset_dims=(), collapsed_slice_dims=(0,), start_index_map=(0,))


def _vperm(v, idx2d):
    return lax.gather(v, idx2d, _GDN, (1,),
                      mode=lax.GatherScatterMode.PROMISE_IN_BOUNDS)


def _edge_logits(xl, xr, src_g, dst_g, att):
    ne = src_g.shape[0]
    per_w = ne // NW
    n_chunks = per_w // _EK
    mesh = plsc.VectorSubcoreMesh(core_axis_name="c", subcore_axis_name="s")

    @functools.partial(
        pl.kernel, mesh=mesh,
        out_type=[
            jax.ShapeDtypeStruct((ne,), jnp.float32),
            jax.ShapeDtypeStruct((NW, 16), jnp.float32),
            jax.ShapeDtypeStruct((ne, H), jnp.float32),
        ],
        scratch_types=[
            pltpu.VMEM((per_w,), jnp.int32),
            pltpu.VMEM((per_w,), jnp.int32),
            pltpu.VMEM((_EK, H), jnp.float32),
            pltpu.VMEM((_EK, H), jnp.float32),
            pltpu.VMEM((_EK, H), jnp.float32),
            pltpu.VMEM((_EK, H), jnp.float32),
            pltpu.VMEM((8, 16), jnp.float32),
            pltpu.VMEM((per_w,), jnp.float32),
            pltpu.VMEM((16,), jnp.float32),
            pltpu.SemaphoreType.DMA,
            pltpu.SemaphoreType.DMA,
            pltpu.SemaphoreType.DMA,
            pltpu.SemaphoreType.DMA,
        ],
    )
    def k(xl_hbm, xr_hbm, src_hbm, dst_hbm, att_hbm, e_hbm, wmax_hbm, gsrc_hbm,
          src_v, dst_v, xla, xra, xlb, xrb, att_v, e_out, gmax_v,
          gsa, gsb, wsa, wsb):
        wid = lax.axis_index("s") * NC + lax.axis_index("c")
        base = wid * per_w
        pltpu.sync_copy(att_hbm, att_v)
        pltpu.sync_copy(src_hbm.at[pl.ds(base, per_w)], src_v)
        pltpu.sync_copy(dst_hbm.at[pl.ds(base, per_w)], dst_v)
        lanes = lax.iota(jnp.int32, 16)
        perms = [(lanes ^ (1 << kk)).reshape(16, 1) for kk in range(4)]

        def g_desc(loc, xlr, xrr, sem):
            return (pltpu.make_async_copy(
                        xl_hbm.at[src_v.at[pl.ds(loc, _EK)]], xlr, sem),
                    pltpu.make_async_copy(
                        xr_hbm.at[dst_v.at[pl.ds(loc, _EK)]], xrr, sem))

        def w_desc(loc, xlr, sem):
            return pltpu.make_async_copy(xlr, gsrc_hbm.at[pl.ds(base + loc, _EK)], sem)

        def issue_g(loc, xlr, xrr, sem):
            a, b = g_desc(loc, xlr, xrr, sem)
            a.start()
            b.start()

        def wait_g(loc, xlr, xrr, sem):
            a, b = g_desc(loc, xlr, xrr, sem)
            a.wait()
            b.wait()

        def compute(loc, xlr, xrr, gmax):
            for grp in range(_EK // 16):
                vs = []
                for jj in range(16):
                    j = grp * 16 + jj
                    acc = jnp.zeros((16,), jnp.float32)
                    for h8 in range(8):
                        m = xlr[j, pl.ds(16 * h8, 16)] + xrr[j, pl.ds(16 * h8, 16)]
                        lr = jnp.maximum(m, 0.2 * m)
                        acc = acc + lr * att_v[h8]
                    vs.append(acc)
                for kk in range(4):
                    sel = ((lanes >> kk) & 1) == 0
                    perm = perms[kk]
                    vs = [jnp.where(sel,
                                    vs[2 * t] + _vperm(vs[2 * t], perm),
                                    vs[2 * t + 1] + _vperm(vs[2 * t + 1], perm))
                          for t in range(len(vs) // 2)]
                e_vec = vs[0]
                e_out[pl.ds(loc + grp * 16, 16)] = e_vec
                gmax = jnp.maximum(gmax, e_vec)
            return gmax

        n_half = n_chunks // 2
        issue_g(0, xla, xra, gsa)

        def body(i, gmax):
            l0 = 2 * i * _EK
            l1 = l0 + _EK
            wait_g(l0, xla, xra, gsa)
            w_desc(l0, xla, wsa).start()

            @pl.when(i > 0)
            def _():
                w_desc(l1, xlb, wsb).wait()
            issue_g(l1, xlb, xrb, gsb)
            gmax = compute(l0, xla, xra, gmax)

            wait_g(l1, xlb, xrb, gsb)
            w_desc(l1, xlb, wsb).start()

            @pl.when(i + 1 < n_half)
            def _():
                w_desc(l0, xla, wsa).wait()
                issue_g(l0 + 2 * _EK, xla, xra, gsa)
            gmax = compute(l1, xlb, xrb, gmax)
            return gmax

        gmax = lax.fori_loop(0, n_half, body,
                             jnp.full((16,), -jnp.inf, jnp.float32))
        w_desc(0, xla, wsa).wait()
        w_desc(0, xlb, wsb).wait()
        gmax_v[...] = gmax
        pltpu.sync_copy(e_out, e_hbm.at[pl.ds(base, per_w)])
        pltpu.sync_copy(gmax_v, wmax_hbm.at[wid])

    return k(xl, xr, src_g, dst_g, att.reshape(8, 16))


_AK = 256


def _agg_body(dst_ref, e_ref, g_ref, gsrc_ref, num_ref, den_ref, num_acc, den_acc):
    j = pl.program_id(1)

    @pl.when(j == 0)
    def _():
        num_acc[...] = jnp.zeros_like(num_acc)
        den_acc[...] = jnp.zeros_like(den_acc)

    w = jnp.exp(e_ref[0, 0] - g_ref[0, 0])
    rows = lax.broadcasted_iota(jnp.int32, (N, _AK), 0)
    oh = jnp.where(rows == dst_ref[0, 0], w, 0.0).astype(jnp.bfloat16)
    gb = gsrc_ref[0].astype(jnp.bfloat16)
    num_acc[...] += jnp.dot(oh, gb, preferred_element_type=jnp.float32)
    den_acc[...] += jnp.dot(oh, jnp.ones((_AK, 1), jnp.bfloat16),
                            preferred_element_type=jnp.float32)

    @pl.when(j == E // _AK - 1)
    def _():
        num_ref[0] = num_acc[...]
        den_ref[0] = den_acc[...]


def _edge_aggregate(dst_l, e_all, g_arr, gsrc):
    return pl.pallas_call(
        _agg_body,
        grid=(NGS, E // _AK),
        in_specs=[
            pl.BlockSpec((1, 1, 1, _AK), lambda i, j: (i, j, 0, 0)),
            pl.BlockSpec((1, 1, 1, _AK), lambda i, j: (i, j, 0, 0)),
            pl.BlockSpec((1, 1), lambda i, j: (0, 0)),
            pl.BlockSpec((1, _AK, H), lambda i, j: (i, j, 0)),
        ],
        out_specs=[
            pl.BlockSpec((1, N, H), lambda i, j: (i, 0, 0)),
            pl.BlockSpec((1, N, 1), lambda i, j: (i, 0, 0)),
        ],
        out_shape=[
            jax.ShapeDtypeStruct((NGS, N, H), jnp.float32),
            jax.ShapeDtypeStruct((NGS, N, 1), jnp.float32),
        ],
        scratch_shapes=[
            pltpu.VMEM((N, H), jnp.float32),
            pltpu.VMEM((N, 1), jnp.float32),
        ],
    )(dst_l.reshape(NGS, E // _AK, 1, _AK),
      e_all.reshape(NGS, E // _AK, 1, _AK),
      g_arr.reshape(1, 16)[:, :1],
      gsrc.reshape(NGS, E, H))


def _proj_body(x_ref, wl_ref, bl_ref, wr_ref, br_ref, xl_ref, xr_ref):
    x = x_ref[...]
    xl_ref[...] = jnp.dot(x, wl_ref[...], preferred_element_type=jnp.float32) + bl_ref[...]
    xr_ref[...] = jnp.dot(x, wr_ref[...], preferred_element_type=jnp.float32) + br_ref[...]


def _projections(x, Wl, bl, Wr, br):
    n = x.shape[0]
    blk = 4096
    grid = (n // blk,)
    return pl.pallas_call(
        _proj_body,
        grid=grid,
        in_specs=[
            pl.BlockSpec((blk, H), lambda i: (i, 0)),
            pl.BlockSpec((H, H), lambda i: (0, 0)),
            pl.BlockSpec((1, H), lambda i: (0, 0)),
            pl.BlockSpec((H, H), lambda i: (0, 0)),
            pl.BlockSpec((1, H), lambda i: (0, 0)),
        ],
        out_specs=[
            pl.BlockSpec((blk, H), lambda i: (i, 0)),
            pl.BlockSpec((blk, H), lambda i: (i, 0)),
        ],
        out_shape=[
            jax.ShapeDtypeStruct((n, H), jnp.float32),
            jax.ShapeDtypeStruct((n, H), jnp.float32),
        ],
    )(x, Wl, bl.reshape(1, H), Wr, br.reshape(1, H))


def _pool_body(num_ref, den_ref, enc_ref, bias_ref, out_ref):
    s = pl.program_id(1)
    den = den_ref[0]
    gat = jnp.where(den > 0.0,
                    num_ref[0] / jnp.where(den > 0.0, den, 1.0), 0.0)
    gat = gat + bias_ref[...] + enc_ref[0]
    colmax = jnp.max(gat, axis=0, keepdims=True)

    @pl.when(s == 0)
    def _():
        out_ref[0] = colmax

    @pl.when(s == 1)
    def _():
        out_ref[0] = jnp.maximum(out_ref[0], colmax)


def _pool(num, den, enc, gat_bias):
    return pl.pallas_call(
        _pool_body,
        grid=(B, 2),
        in_specs=[
            pl.BlockSpec((1, N, H), lambda b, s: (s * B + b, 0, 0)),
            pl.BlockSpec((1, N, 1), lambda b, s: (s * B + b, 0, 0)),
            pl.BlockSpec((1, N, H), lambda b, s: (s * B + b, 0, 0)),
            pl.BlockSpec((1, H), lambda b, s: (0, 0)),
        ],
        out_specs=pl.BlockSpec((1, 1, H), lambda b, s: (b, 0, 0)),
        out_shape=jax.ShapeDtypeStruct((B, 1, H), jnp.float32),
    )(num, den, enc, gat_bias.reshape(1, H)).reshape(B, H)


def _mlp_body(fused_ref, W0_ref, b0_ref, W1_ref, b1_ref, W2_ref, b2_ref, out_ref):
    h = jnp.maximum(jnp.dot(fused_ref[...], W0_ref[...],
                            preferred_element_type=jnp.float32) + b0_ref[...], 0.0)
    h = jnp.maximum(jnp.dot(h, W1_ref[...],
                            preferred_element_type=jnp.float32) + b1_ref[...], 0.0)
    out_ref[...] = jnp.dot(h, W2_ref[...],
                           preferred_element_type=jnp.float32) + b2_ref[...]


def _mlp_head(fused, W0, b0, W1, b1, W2, b2):
    return pl.pallas_call(
        _mlp_body,
        out_shape=jax.ShapeDtypeStruct((B, C), jnp.float32),
    )(fused, W0, b0.reshape(1, -1), W1, b1.reshape(1, -1), W2, b2.reshape(1, -1))


def kernel(diff_input, msg_input, graph_edge_index_diff, graph_edge_index_msg,
           emb, Wl, bl, Wr, br, att, gat_bias, W0, b0, W1, b1, W2, b2):
    ids_all = jnp.concatenate(
        [diff_input[0].reshape(-1), msg_input[0].reshape(-1)]).astype(jnp.int32)
    enc_all = _gather_rows(emb, ids_all)

    xl, xr = _projections(enc_all, Wl, bl, Wr, br)

    src_l = jnp.concatenate(
        [graph_edge_index_diff[:, 0, :], graph_edge_index_msg[:, 0, :]]
    ).astype(jnp.int32)
    dst_l = jnp.concatenate(
        [graph_edge_index_diff[:, 1, :], graph_edge_index_msg[:, 1, :]]
    ).astype(jnp.int32)
    base = (jnp.arange(NGS, dtype=jnp.int32) * N)[:, None]
    src_g = (src_l + base).reshape(-1)
    dst_g = (dst_l + base).reshape(-1)

    e_all, wmax, gsrc = _edge_logits(xl, xr, src_g, dst_g, att)
    g_arr = jnp.full((16,), jnp.max(wmax), jnp.float32)
    num, den = _edge_aggregate(dst_l, e_all, g_arr, gsrc)

    enc3 = enc_all.reshape(NGS, N, H)
    fused = _pool(num, den, enc3, gat_bias)
    return _mlp_head(fused, W0, b0, W1, b1, W2, b2)


# speedup vs baseline: 1.2548x; 1.2548x over previous
import functools

import jax
import jax.numpy as jnp
from jax import lax
from jax.experimental import pallas as pl
from jax.experimental.pallas import tpu as pltpu
from jax.experimental.pallas import tpu_sc as plsc

B, N, E, H, V, C = 8, 2048, 32768, 128, 50000, 2
NGS = 2 * B

_info = plsc.get_sparse_core_info()
NC, NS = _info.num_cores, _info.num_subcores
NW = NC * NS


def _gather_rows(table, idx):
    n = idx.shape[0]
    per_w = n // NW
    chunk = 512
    mesh = plsc.VectorSubcoreMesh(core_axis_name="c", subcore_axis_name="s")

    @functools.partial(
        pl.kernel, mesh=mesh,
        out_type=jax.ShapeDtypeStruct((n, H), jnp.float32),
        scratch_types=[
            pltpu.VMEM((chunk,), jnp.int32),
            pltpu.VMEM((chunk, H), jnp.float32),
            pltpu.SemaphoreType.DMA,
        ],
    )
    def k(table_hbm, idx_hbm, out_hbm, idx_v, rows_v, sem):
        wid = lax.axis_index("s") * NC + lax.axis_index("c")
        base = wid * per_w

        def body(i, carry):
            off = base + i * chunk
            pltpu.sync_copy(idx_hbm.at[pl.ds(off, chunk)], idx_v)
            pltpu.async_copy(table_hbm.at[idx_v], rows_v, sem).wait()
            pltpu.sync_copy(rows_v, out_hbm.at[pl.ds(off, chunk)])
            return carry

        lax.fori_loop(0, per_w // chunk, body, 0)

    return k(table, idx)


_EK = 32

_GDN = lax.GatherDimensionNumbers(
    offset_dims=(), collapsed_slice_dims=(0,), start_index_map=(0,))


def _vperm(v, idx2d):
    return lax.gather(v, idx2d, _GDN, (1,),
                      mode=lax.GatherScatterMode.PROMISE_IN_BOUNDS)


def _edge_logits(xl, xr, src_g, dst_g, att):
    ne = src_g.shape[0]
    per_w = ne // NW
    n_chunks = per_w // _EK
    mesh = plsc.VectorSubcoreMesh(core_axis_name="c", subcore_axis_name="s")

    @functools.partial(
        pl.kernel, mesh=mesh,
        out_type=[
            jax.ShapeDtypeStruct((ne,), jnp.float32),
            jax.ShapeDtypeStruct((NW, 16), jnp.float32),
            jax.ShapeDtypeStruct((ne, H), jnp.float32),
        ],
        scratch_types=[
            pltpu.VMEM((per_w,), jnp.int32),
            pltpu.VMEM((per_w,), jnp.int32),
            pltpu.VMEM((_EK, H), jnp.float32),
            pltpu.VMEM((_EK, H), jnp.float32),
            pltpu.VMEM((_EK, H), jnp.float32),
            pltpu.VMEM((_EK, H), jnp.float32),
            pltpu.VMEM((8, 16), jnp.float32),
            pltpu.VMEM((per_w,), jnp.float32),
            pltpu.VMEM((16,), jnp.float32),
            pltpu.SemaphoreType.DMA,
            pltpu.SemaphoreType.DMA,
            pltpu.SemaphoreType.DMA,
            pltpu.SemaphoreType.DMA,
        ],
    )
    def k(xl_hbm, xr_hbm, src_hbm, dst_hbm, att_hbm, e_hbm, wmax_hbm, gsrc_hbm,
          src_v, dst_v, xla, xra, xlb, xrb, att_v, e_out, gmax_v,
          gsa, gsb, wsa, wsb):
        wid = lax.axis_index("s") * NC + lax.axis_index("c")
        base = wid * per_w
        pltpu.sync_copy(att_hbm, att_v)
        pltpu.sync_copy(src_hbm.at[pl.ds(base, per_w)], src_v)
        pltpu.sync_copy(dst_hbm.at[pl.ds(base, per_w)], dst_v)
        lanes = lax.iota(jnp.int32, 16)
        perms = [(lanes ^ (1 << kk)).reshape(16, 1) for kk in range(4)]

        def g_desc(loc, xlr, xrr, sem):
            return (pltpu.make_async_copy(
                        xl_hbm.at[src_v.at[pl.ds(loc, _EK)]], xlr, sem),
                    pltpu.make_async_copy(
                        xr_hbm.at[dst_v.at[pl.ds(loc, _EK)]], xrr, sem))

        def w_desc(loc, xlr, sem):
            return pltpu.make_async_copy(xlr, gsrc_hbm.at[pl.ds(base + loc, _EK)], sem)

        def issue_g(loc, xlr, xrr, sem):
            a, b = g_desc(loc, xlr, xrr, sem)
            a.start()
            b.start()

        def wait_g(loc, xlr, xrr, sem):
            a, b = g_desc(loc, xlr, xrr, sem)
            a.wait()
            b.wait()

        def compute(loc, xlr, xrr, gmax):
            for grp in range(_EK // 16):
                vs = []
                for jj in range(16):
                    j = grp * 16 + jj
                    acc = jnp.zeros((16,), jnp.float32)
                    for h8 in range(8):
                        m = xlr[j, pl.ds(16 * h8, 16)] + xrr[j, pl.ds(16 * h8, 16)]
                        lr = jnp.maximum(m, 0.2 * m)
                        acc = acc + lr * att_v[h8]
                    vs.append(acc)
                for kk in range(4):
                    sel = ((lanes >> kk) & 1) == 0
                    perm = perms[kk]
                    vs = [jnp.where(sel,
                                    vs[2 * t] + _vperm(vs[2 * t], perm),
                                    vs[2 * t + 1] + _vperm(vs[2 * t + 1], perm))
                          for t in range(len(vs) // 2)]
                e_vec = vs[0]
                e_out[pl.ds(loc + grp * 16, 16)] = e_vec
                gmax = jnp.maximum(gmax, e_vec)
            return gmax

        n_half = n_chunks // 2
        issue_g(0, xla, xra, gsa)

        def body(i, gmax):
            l0 = 2 * i * _EK
            l1 = l0 + _EK
            wait_g(l0, xla, xra, gsa)
            w_desc(l0, xla, wsa).start()

            @pl.when(i > 0)
            def _():
                w_desc(l1, xlb, wsb).wait()
            issue_g(l1, xlb, xrb, gsb)
            gmax = compute(l0, xla, xra, gmax)

            wait_g(l1, xlb, xrb, gsb)
            w_desc(l1, xlb, wsb).start()

            @pl.when(i + 1 < n_half)
            def _():
                w_desc(l0, xla, wsa).wait()
                issue_g(l0 + 2 * _EK, xla, xra, gsa)
            gmax = compute(l1, xlb, xrb, gmax)
            return gmax

        gmax = lax.fori_loop(0, n_half, body,
                             jnp.full((16,), -jnp.inf, jnp.float32))
        w_desc(0, xla, wsa).wait()
        w_desc(0, xlb, wsb).wait()
        gmax_v[...] = gmax
        pltpu.sync_copy(e_out, e_hbm.at[pl.ds(base, per_w)])
        pltpu.sync_copy(gmax_v, wmax_hbm.at[wid])

    return k(xl, xr, src_g, dst_g, att.reshape(8, 16))


_AK = 1024


def _agg_body(dst_ref, e_ref, g_ref, gsrc_ref, num_ref, den_ref, num_acc, den_acc):
    j = pl.program_id(1)

    @pl.when(j == 0)
    def _():
        num_acc[...] = jnp.zeros_like(num_acc)
        den_acc[...] = jnp.zeros_like(den_acc)

    w = jnp.exp(e_ref[0, 0] - g_ref[0, 0])
    rows = lax.broadcasted_iota(jnp.int32, (N, _AK), 0)
    oh = jnp.where(rows == dst_ref[0, 0], w, 0.0).astype(jnp.bfloat16)
    gb = gsrc_ref[0].astype(jnp.bfloat16)
    num_acc[...] += jnp.dot(oh, gb, preferred_element_type=jnp.float32)
    den_acc[...] += jnp.dot(oh, jnp.ones((_AK, 1), jnp.bfloat16),
                            preferred_element_type=jnp.float32)

    @pl.when(j == E // _AK - 1)
    def _():
        num_ref[0] = num_acc[...]
        den_ref[0] = den_acc[...]


def _edge_aggregate(dst_l, e_all, g_arr, gsrc):
    return pl.pallas_call(
        _agg_body,
        grid=(NGS, E // _AK),
        in_specs=[
            pl.BlockSpec((1, 1, 1, _AK), lambda i, j: (i, j, 0, 0)),
            pl.BlockSpec((1, 1, 1, _AK), lambda i, j: (i, j, 0, 0)),
            pl.BlockSpec((1, 1), lambda i, j: (0, 0)),
            pl.BlockSpec((1, _AK, H), lambda i, j: (i, j, 0)),
        ],
        out_specs=[
            pl.BlockSpec((1, N, H), lambda i, j: (i, 0, 0)),
            pl.BlockSpec((1, N, 1), lambda i, j: (i, 0, 0)),
        ],
        out_shape=[
            jax.ShapeDtypeStruct((NGS, N, H), jnp.float32),
            jax.ShapeDtypeStruct((NGS, N, 1), jnp.float32),
        ],
        scratch_shapes=[
            pltpu.VMEM((N, H), jnp.float32),
            pltpu.VMEM((N, 1), jnp.float32),
        ],
    )(dst_l.reshape(NGS, E // _AK, 1, _AK),
      e_all.reshape(NGS, E // _AK, 1, _AK),
      g_arr.reshape(1, 16)[:, :1],
      gsrc.reshape(NGS, E, H))


def _proj_body(x_ref, wl_ref, bl_ref, wr_ref, br_ref, xl_ref, xr_ref):
    x = x_ref[...]
    xl_ref[...] = jnp.dot(x, wl_ref[...], preferred_element_type=jnp.float32) + bl_ref[...]
    xr_ref[...] = jnp.dot(x, wr_ref[...], preferred_element_type=jnp.float32) + br_ref[...]


def _projections(x, Wl, bl, Wr, br):
    n = x.shape[0]
    blk = 4096
    grid = (n // blk,)
    return pl.pallas_call(
        _proj_body,
        grid=grid,
        in_specs=[
            pl.BlockSpec((blk, H), lambda i: (i, 0)),
            pl.BlockSpec((H, H), lambda i: (0, 0)),
            pl.BlockSpec((1, H), lambda i: (0, 0)),
            pl.BlockSpec((H, H), lambda i: (0, 0)),
            pl.BlockSpec((1, H), lambda i: (0, 0)),
        ],
        out_specs=[
            pl.BlockSpec((blk, H), lambda i: (i, 0)),
            pl.BlockSpec((blk, H), lambda i: (i, 0)),
        ],
        out_shape=[
            jax.ShapeDtypeStruct((n, H), jnp.float32),
            jax.ShapeDtypeStruct((n, H), jnp.float32),
        ],
    )(x, Wl, bl.reshape(1, H), Wr, br.reshape(1, H))


def _pool_body(num_ref, den_ref, enc_ref, bias_ref, out_ref):
    s = pl.program_id(1)
    den = den_ref[0]
    gat = jnp.where(den > 0.0,
                    num_ref[0] / jnp.where(den > 0.0, den, 1.0), 0.0)
    gat = gat + bias_ref[...] + enc_ref[0]
    colmax = jnp.max(gat, axis=0, keepdims=True)

    @pl.when(s == 0)
    def _():
        out_ref[0] = colmax

    @pl.when(s == 1)
    def _():
        out_ref[0] = jnp.maximum(out_ref[0], colmax)


def _pool(num, den, enc, gat_bias):
    return pl.pallas_call(
        _pool_body,
        grid=(B, 2),
        in_specs=[
            pl.BlockSpec((1, N, H), lambda b, s: (s * B + b, 0, 0)),
            pl.BlockSpec((1, N, 1), lambda b, s: (s * B + b, 0, 0)),
            pl.BlockSpec((1, N, H), lambda b, s: (s * B + b, 0, 0)),
            pl.BlockSpec((1, H), lambda b, s: (0, 0)),
        ],
        out_specs=pl.BlockSpec((1, 1, H), lambda b, s: (b, 0, 0)),
        out_shape=jax.ShapeDtypeStruct((B, 1, H), jnp.float32),
    )(num, den, enc, gat_bias.reshape(1, H)).reshape(B, H)


def _mlp_body(fused_ref, W0_ref, b0_ref, W1_ref, b1_ref, W2_ref, b2_ref, out_ref):
    h = jnp.maximum(jnp.dot(fused_ref[...], W0_ref[...],
                            preferred_element_type=jnp.float32) + b0_ref[...], 0.0)
    h = jnp.maximum(jnp.dot(h, W1_ref[...],
                            preferred_element_type=jnp.float32) + b1_ref[...], 0.0)
    out_ref[...] = jnp.dot(h, W2_ref[...],
                           preferred_element_type=jnp.float32) + b2_ref[...]


def _mlp_head(fused, W0, b0, W1, b1, W2, b2):
    return pl.pallas_call(
        _mlp_body,
        out_shape=jax.ShapeDtypeStruct((B, C), jnp.float32),
    )(fused, W0, b0.reshape(1, -1), W1, b1.reshape(1, -1), W2, b2.reshape(1, -1))


def kernel(diff_input, msg_input, graph_edge_index_diff, graph_edge_index_msg,
           emb, Wl, bl, Wr, br, att, gat_bias, W0, b0, W1, b1, W2, b2):
    ids_all = jnp.concatenate(
        [diff_input[0].reshape(-1), msg_input[0].reshape(-1)]).astype(jnp.int32)
    enc_all = _gather_rows(emb, ids_all)

    xl, xr = _projections(enc_all, Wl, bl, Wr, br)

    src_l = jnp.concatenate(
        [graph_edge_index_diff[:, 0, :], graph_edge_index_msg[:, 0, :]]
    ).astype(jnp.int32)
    dst_l = jnp.concatenate(
        [graph_edge_index_diff[:, 1, :], graph_edge_index_msg[:, 1, :]]
    ).astype(jnp.int32)
    base = (jnp.arange(NGS, dtype=jnp.int32) * N)[:, None]
    src_g = (src_l + base).reshape(-1)
    dst_g = (dst_l + base).reshape(-1)

    e_all, wmax, gsrc = _edge_logits(xl, xr, src_g, dst_g, att)
    g_arr = jnp.full((16,), jnp.max(wmax), jnp.float32)
    num, den = _edge_aggregate(dst_l, e_all, g_arr, gsrc)

    enc3 = enc_all.reshape(NGS, N, H)
    fused = _pool(num, den, enc3, gat_bias)
    return _mlp_head(fused, W0, b0, W1, b1, W2, b2)


# speedup vs baseline: 1.2955x; 1.0324x over previous
import functools

import jax
import jax.numpy as jnp
from jax import lax
from jax.experimental import pallas as pl
from jax.experimental.pallas import tpu as pltpu
from jax.experimental.pallas import tpu_sc as plsc

B, N, E, H, V, C = 8, 2048, 32768, 128, 50000, 2
NGS = 2 * B

_info = plsc.get_sparse_core_info()
NC, NS = _info.num_cores, _info.num_subcores
NW = NC * NS


def _gather_rows(table, idx):
    n = idx.shape[0]
    per_w = n // NW
    chunk = 512
    mesh = plsc.VectorSubcoreMesh(core_axis_name="c", subcore_axis_name="s")

    @functools.partial(
        pl.kernel, mesh=mesh,
        out_type=jax.ShapeDtypeStruct((n, H), jnp.float32),
        scratch_types=[
            pltpu.VMEM((chunk,), jnp.int32),
            pltpu.VMEM((chunk, H), jnp.float32),
            pltpu.SemaphoreType.DMA,
        ],
    )
    def k(table_hbm, idx_hbm, out_hbm, idx_v, rows_v, sem):
        wid = lax.axis_index("s") * NC + lax.axis_index("c")
        base = wid * per_w

        def body(i, carry):
            off = base + i * chunk
            pltpu.sync_copy(idx_hbm.at[pl.ds(off, chunk)], idx_v)
            pltpu.async_copy(table_hbm.at[idx_v], rows_v, sem).wait()
            pltpu.sync_copy(rows_v, out_hbm.at[pl.ds(off, chunk)])
            return carry

        lax.fori_loop(0, per_w // chunk, body, 0)

    return k(table, idx)


_EK = 32

_GDN = lax.GatherDimensionNumbers(
    offset_dims=(), collapsed_slice_dims=(0,), start_index_map=(0,))


def _vperm(v, idx2d):
    return lax.gather(v, idx2d, _GDN, (1,),
                      mode=lax.GatherScatterMode.PROMISE_IN_BOUNDS)


def _edge_logits(xl, xr, src_g, dst_g, att):
    ne = src_g.shape[0]
    per_w = ne // NW
    n_chunks = per_w // _EK
    mesh = plsc.VectorSubcoreMesh(core_axis_name="c", subcore_axis_name="s")

    @functools.partial(
        pl.kernel, mesh=mesh,
        out_type=[
            jax.ShapeDtypeStruct((ne,), jnp.float32),
            jax.ShapeDtypeStruct((NW, 16), jnp.float32),
            jax.ShapeDtypeStruct((ne, H), jnp.float32),
        ],
        scratch_types=[
            pltpu.VMEM((per_w,), jnp.int32),
            pltpu.VMEM((per_w,), jnp.int32),
            pltpu.VMEM((_EK, H), jnp.float32),
            pltpu.VMEM((_EK, H), jnp.float32),
            pltpu.VMEM((_EK, H), jnp.float32),
            pltpu.VMEM((_EK, H), jnp.float32),
            pltpu.VMEM((8, 16), jnp.float32),
            pltpu.VMEM((per_w,), jnp.float32),
            pltpu.VMEM((16,), jnp.float32),
            pltpu.SemaphoreType.DMA,
            pltpu.SemaphoreType.DMA,
            pltpu.SemaphoreType.DMA,
            pltpu.SemaphoreType.DMA,
        ],
    )
    def k(xl_hbm, xr_hbm, src_hbm, dst_hbm, att_hbm, e_hbm, wmax_hbm, gsrc_hbm,
          src_v, dst_v, xla, xra, xlb, xrb, att_v, e_out, gmax_v,
          gsa, gsb, wsa, wsb):
        wid = lax.axis_index("s") * NC + lax.axis_index("c")
        base = wid * per_w
        pltpu.sync_copy(att_hbm, att_v)
        pltpu.sync_copy(src_hbm.at[pl.ds(base, per_w)], src_v)
        pltpu.sync_copy(dst_hbm.at[pl.ds(base, per_w)], dst_v)
        lanes = lax.iota(jnp.int32, 16)
        perms = [(lanes ^ (1 << kk)).reshape(16, 1) for kk in range(4)]

        def g_desc(loc, xlr, xrr, sem):
            return (pltpu.make_async_copy(
                        xl_hbm.at[src_v.at[pl.ds(loc, _EK)]], xlr, sem),
                    pltpu.make_async_copy(
                        xr_hbm.at[dst_v.at[pl.ds(loc, _EK)]], xrr, sem))

        def w_desc(loc, xlr, sem):
            return pltpu.make_async_copy(xlr, gsrc_hbm.at[pl.ds(base + loc, _EK)], sem)

        def issue_g(loc, xlr, xrr, sem):
            a, b = g_desc(loc, xlr, xrr, sem)
            a.start()
            b.start()

        def wait_g(loc, xlr, xrr, sem):
            a, b = g_desc(loc, xlr, xrr, sem)
            a.wait()
            b.wait()

        def compute(loc, xlr, xrr, gmax):
            for grp in range(_EK // 16):
                vs = []
                for jj in range(16):
                    j = grp * 16 + jj
                    acc = jnp.zeros((16,), jnp.float32)
                    for h8 in range(8):
                        m = xlr[j, pl.ds(16 * h8, 16)] + xrr[j, pl.ds(16 * h8, 16)]
                        lr = jnp.maximum(m, 0.2 * m)
                        acc = acc + lr * att_v[h8]
                    vs.append(acc)
                for kk in range(4):
                    sel = ((lanes >> kk) & 1) == 0
                    perm = perms[kk]
                    vs = [jnp.where(sel,
                                    vs[2 * t] + _vperm(vs[2 * t], perm),
                                    vs[2 * t + 1] + _vperm(vs[2 * t + 1], perm))
                          for t in range(len(vs) // 2)]
                e_vec = vs[0]
                e_out[pl.ds(loc + grp * 16, 16)] = e_vec
                gmax = jnp.maximum(gmax, e_vec)
            return gmax

        n_half = n_chunks // 2
        issue_g(0, xla, xra, gsa)

        def body(i, gmax):
            l0 = 2 * i * _EK
            l1 = l0 + _EK
            wait_g(l0, xla, xra, gsa)
            w_desc(l0, xla, wsa).start()

            @pl.when(i > 0)
            def _():
                w_desc(l1, xlb, wsb).wait()
            issue_g(l1, xlb, xrb, gsb)
            gmax = compute(l0, xla, xra, gmax)

            wait_g(l1, xlb, xrb, gsb)
            w_desc(l1, xlb, wsb).start()

            @pl.when(i + 1 < n_half)
            def _():
                w_desc(l0, xla, wsa).wait()
                issue_g(l0 + 2 * _EK, xla, xra, gsa)
            gmax = compute(l1, xlb, xrb, gmax)
            return gmax

        gmax = lax.fori_loop(0, n_half, body,
                             jnp.full((16,), -jnp.inf, jnp.float32))
        w_desc(0, xla, wsa).wait()
        w_desc(0, xlb, wsb).wait()
        gmax_v[...] = gmax
        pltpu.sync_copy(e_out, e_hbm.at[pl.ds(base, per_w)])
        pltpu.sync_copy(gmax_v, wmax_hbm.at[wid])

    return k(xl, xr, src_g, dst_g, att.reshape(8, 16))


_AK = 2048


def _agg_body(dst_ref, e_ref, g_ref, gsrc_ref, num_ref, den_ref, num_acc, den_acc):
    j = pl.program_id(1)

    @pl.when(j == 0)
    def _():
        num_acc[...] = jnp.zeros_like(num_acc)
        den_acc[...] = jnp.zeros_like(den_acc)

    w = jnp.exp(e_ref[0, 0] - g_ref[0, 0])
    rows = lax.broadcasted_iota(jnp.int32, (N, _AK), 0)
    oh = jnp.where(rows == dst_ref[0, 0], w, 0.0).astype(jnp.bfloat16)
    gb = gsrc_ref[0].astype(jnp.bfloat16)
    num_acc[...] += jnp.dot(oh, gb, preferred_element_type=jnp.float32)
    den_acc[...] += jnp.dot(oh, jnp.ones((_AK, 1), jnp.bfloat16),
                            preferred_element_type=jnp.float32)

    @pl.when(j == E // _AK - 1)
    def _():
        num_ref[0] = num_acc[...]
        den_ref[0] = den_acc[...]


def _edge_aggregate(dst_l, e_all, g_arr, gsrc):
    return pl.pallas_call(
        _agg_body,
        grid=(NGS, E // _AK),
        in_specs=[
            pl.BlockSpec((1, 1, 1, _AK), lambda i, j: (i, j, 0, 0)),
            pl.BlockSpec((1, 1, 1, _AK), lambda i, j: (i, j, 0, 0)),
            pl.BlockSpec((1, 1), lambda i, j: (0, 0)),
            pl.BlockSpec((1, _AK, H), lambda i, j: (i, j, 0)),
        ],
        out_specs=[
            pl.BlockSpec((1, N, H), lambda i, j: (i, 0, 0)),
            pl.BlockSpec((1, N, 1), lambda i, j: (i, 0, 0)),
        ],
        out_shape=[
            jax.ShapeDtypeStruct((NGS, N, H), jnp.float32),
            jax.ShapeDtypeStruct((NGS, N, 1), jnp.float32),
        ],
        scratch_shapes=[
            pltpu.VMEM((N, H), jnp.float32),
            pltpu.VMEM((N, 1), jnp.float32),
        ],
    )(dst_l.reshape(NGS, E // _AK, 1, _AK),
      e_all.reshape(NGS, E // _AK, 1, _AK),
      g_arr.reshape(1, 16)[:, :1],
      gsrc.reshape(NGS, E, H))


def _proj_body(x_ref, wl_ref, bl_ref, wr_ref, br_ref, xl_ref, xr_ref):
    x = x_ref[...]
    xl_ref[...] = jnp.dot(x, wl_ref[...], preferred_element_type=jnp.float32) + bl_ref[...]
    xr_ref[...] = jnp.dot(x, wr_ref[...], preferred_element_type=jnp.float32) + br_ref[...]


def _projections(x, Wl, bl, Wr, br):
    n = x.shape[0]
    blk = 4096
    grid = (n // blk,)
    return pl.pallas_call(
        _proj_body,
        grid=grid,
        in_specs=[
            pl.BlockSpec((blk, H), lambda i: (i, 0)),
            pl.BlockSpec((H, H), lambda i: (0, 0)),
            pl.BlockSpec((1, H), lambda i: (0, 0)),
            pl.BlockSpec((H, H), lambda i: (0, 0)),
            pl.BlockSpec((1, H), lambda i: (0, 0)),
        ],
        out_specs=[
            pl.BlockSpec((blk, H), lambda i: (i, 0)),
            pl.BlockSpec((blk, H), lambda i: (i, 0)),
        ],
        out_shape=[
            jax.ShapeDtypeStruct((n, H), jnp.float32),
            jax.ShapeDtypeStruct((n, H), jnp.float32),
        ],
    )(x, Wl, bl.reshape(1, H), Wr, br.reshape(1, H))


def _pool_body(num_ref, den_ref, enc_ref, bias_ref, out_ref):
    s = pl.program_id(1)
    den = den_ref[0]
    gat = jnp.where(den > 0.0,
                    num_ref[0] / jnp.where(den > 0.0, den, 1.0), 0.0)
    gat = gat + bias_ref[...] + enc_ref[0]
    colmax = jnp.max(gat, axis=0, keepdims=True)

    @pl.when(s == 0)
    def _():
        out_ref[0] = colmax

    @pl.when(s == 1)
    def _():
        out_ref[0] = jnp.maximum(out_ref[0], colmax)


def _pool(num, den, enc, gat_bias):
    return pl.pallas_call(
        _pool_body,
        grid=(B, 2),
        in_specs=[
            pl.BlockSpec((1, N, H), lambda b, s: (s * B + b, 0, 0)),
            pl.BlockSpec((1, N, 1), lambda b, s: (s * B + b, 0, 0)),
            pl.BlockSpec((1, N, H), lambda b, s: (s * B + b, 0, 0)),
            pl.BlockSpec((1, H), lambda b, s: (0, 0)),
        ],
        out_specs=pl.BlockSpec((1, 1, H), lambda b, s: (b, 0, 0)),
        out_shape=jax.ShapeDtypeStruct((B, 1, H), jnp.float32),
    )(num, den, enc, gat_bias.reshape(1, H)).reshape(B, H)


def _mlp_body(fused_ref, W0_ref, b0_ref, W1_ref, b1_ref, W2_ref, b2_ref, out_ref):
    h = jnp.maximum(jnp.dot(fused_ref[...], W0_ref[...],
                            preferred_element_type=jnp.float32) + b0_ref[...], 0.0)
    h = jnp.maximum(jnp.dot(h, W1_ref[...],
                            preferred_element_type=jnp.float32) + b1_ref[...], 0.0)
    out_ref[...] = jnp.dot(h, W2_ref[...],
                           preferred_element_type=jnp.float32) + b2_ref[...]


def _mlp_head(fused, W0, b0, W1, b1, W2, b2):
    return pl.pallas_call(
        _mlp_body,
        out_shape=jax.ShapeDtypeStruct((B, C), jnp.float32),
    )(fused, W0, b0.reshape(1, -1), W1, b1.reshape(1, -1), W2, b2.reshape(1, -1))


def kernel(diff_input, msg_input, graph_edge_index_diff, graph_edge_index_msg,
           emb, Wl, bl, Wr, br, att, gat_bias, W0, b0, W1, b1, W2, b2):
    ids_all = jnp.concatenate(
        [diff_input[0].reshape(-1), msg_input[0].reshape(-1)]).astype(jnp.int32)
    enc_all = _gather_rows(emb, ids_all)

    xl, xr = _projections(enc_all, Wl, bl, Wr, br)

    src_l = jnp.concatenate(
        [graph_edge_index_diff[:, 0, :], graph_edge_index_msg[:, 0, :]]
    ).astype(jnp.int32)
    dst_l = jnp.concatenate(
        [graph_edge_index_diff[:, 1, :], graph_edge_index_msg[:, 1, :]]
    ).astype(jnp.int32)
    base = (jnp.arange(NGS, dtype=jnp.int32) * N)[:, None]
    src_g = (src_l + base).reshape(-1)
    dst_g = (dst_l + base).reshape(-1)

    e_all, wmax, gsrc = _edge_logits(xl, xr, src_g, dst_g, att)
    g_arr = jnp.full((16,), jnp.max(wmax), jnp.float32)
    num, den = _edge_aggregate(dst_l, e_all, g_arr, gsrc)

    enc3 = enc_all.reshape(NGS, N, H)
    fused = _pool(num, den, enc3, gat_bias)
    return _mlp_head(fused, W0, b0, W1, b1, W2, b2)


# speedup vs baseline: 1.5047x; 1.1615x over previous
import functools

import jax
import jax.numpy as jnp
from jax import lax
from jax.experimental import pallas as pl
from jax.experimental.pallas import tpu as pltpu
from jax.experimental.pallas import tpu_sc as plsc

B, N, E, H, V, C = 8, 2048, 32768, 128, 50000, 2
NGS = 2 * B

_info = plsc.get_sparse_core_info()
NC, NS = _info.num_cores, _info.num_subcores
NW = NC * NS


def _gather_rows(table, idx):
    n = idx.shape[0]
    per_w = n // NW
    chunk = 512
    mesh = plsc.VectorSubcoreMesh(core_axis_name="c", subcore_axis_name="s")

    @functools.partial(
        pl.kernel, mesh=mesh,
        out_type=jax.ShapeDtypeStruct((n, H), jnp.float32),
        scratch_types=[
            pltpu.VMEM((chunk,), jnp.int32),
            pltpu.VMEM((chunk, H), jnp.float32),
            pltpu.SemaphoreType.DMA,
        ],
    )
    def k(table_hbm, idx_hbm, out_hbm, idx_v, rows_v, sem):
        wid = lax.axis_index("s") * NC + lax.axis_index("c")
        base = wid * per_w

        def body(i, carry):
            off = base + i * chunk
            pltpu.sync_copy(idx_hbm.at[pl.ds(off, chunk)], idx_v)
            pltpu.async_copy(table_hbm.at[idx_v], rows_v, sem).wait()
            pltpu.sync_copy(rows_v, out_hbm.at[pl.ds(off, chunk)])
            return carry

        lax.fori_loop(0, per_w // chunk, body, 0)

    return k(table, idx)


_EK = 32

_GDN = lax.GatherDimensionNumbers(
    offset_dims=(), collapsed_slice_dims=(0,), start_index_map=(0,))


def _vperm(v, idx2d):
    return lax.gather(v, idx2d, _GDN, (1,),
                      mode=lax.GatherScatterMode.PROMISE_IN_BOUNDS)


def _edge_logits(xl, xr, src_g, dst_g, att):
    ne = src_g.shape[0]
    per_w = ne // NW
    n_chunks = per_w // _EK
    mesh = plsc.VectorSubcoreMesh(core_axis_name="c", subcore_axis_name="s")

    @functools.partial(
        pl.kernel, mesh=mesh,
        out_type=[
            jax.ShapeDtypeStruct((ne,), jnp.float32),
            jax.ShapeDtypeStruct((NW, 16), jnp.float32),
            jax.ShapeDtypeStruct((ne, H), jnp.float32),
        ],
        scratch_types=[
            pltpu.VMEM((per_w,), jnp.int32),
            pltpu.VMEM((per_w,), jnp.int32),
            pltpu.VMEM((_EK, H), jnp.float32),
            pltpu.VMEM((_EK, H), jnp.float32),
            pltpu.VMEM((_EK, H), jnp.float32),
            pltpu.VMEM((_EK, H), jnp.float32),
            pltpu.VMEM((8, 16), jnp.float32),
            pltpu.VMEM((per_w,), jnp.float32),
            pltpu.VMEM((16,), jnp.float32),
            pltpu.SemaphoreType.DMA,
            pltpu.SemaphoreType.DMA,
            pltpu.SemaphoreType.DMA,
            pltpu.SemaphoreType.DMA,
        ],
    )
    def k(xl_hbm, xr_hbm, src_hbm, dst_hbm, att_hbm, e_hbm, wmax_hbm, gsrc_hbm,
          src_v, dst_v, xla, xra, xlb, xrb, att_v, e_out, gmax_v,
          gsa, gsb, wsa, wsb):
        wid = lax.axis_index("s") * NC + lax.axis_index("c")
        base = wid * per_w
        pltpu.sync_copy(att_hbm, att_v)
        pltpu.sync_copy(src_hbm.at[pl.ds(base, per_w)], src_v)
        pltpu.sync_copy(dst_hbm.at[pl.ds(base, per_w)], dst_v)
        lanes = lax.iota(jnp.int32, 16)
        perms = [(lanes ^ (1 << kk)).reshape(16, 1) for kk in range(4)]

        def g_desc(loc, xlr, xrr, sem):
            return (pltpu.make_async_copy(
                        xl_hbm.at[src_v.at[pl.ds(loc, _EK)]], xlr, sem),
                    pltpu.make_async_copy(
                        xr_hbm.at[dst_v.at[pl.ds(loc, _EK)]], xrr, sem))

        def w_desc(loc, xlr, sem):
            return pltpu.make_async_copy(xlr, gsrc_hbm.at[pl.ds(base + loc, _EK)], sem)

        def issue_g(loc, xlr, xrr, sem):
            a, b = g_desc(loc, xlr, xrr, sem)
            a.start()
            b.start()

        def wait_g(loc, xlr, xrr, sem):
            a, b = g_desc(loc, xlr, xrr, sem)
            a.wait()
            b.wait()

        def compute(loc, xlr, xrr, gmax):
            for grp in range(_EK // 16):
                vs = []
                for jj in range(16):
                    j = grp * 16 + jj
                    acc = jnp.zeros((16,), jnp.float32)
                    for h8 in range(8):
                        m = xlr[j, pl.ds(16 * h8, 16)] + xrr[j, pl.ds(16 * h8, 16)]
                        lr = jnp.maximum(m, 0.2 * m)
                        acc = acc + lr * att_v[h8]
                    vs.append(acc)
                for kk in range(4):
                    sel = ((lanes >> kk) & 1) == 0
                    perm = perms[kk]
                    vs = [jnp.where(sel,
                                    vs[2 * t] + _vperm(vs[2 * t], perm),
                                    vs[2 * t + 1] + _vperm(vs[2 * t + 1], perm))
                          for t in range(len(vs) // 2)]
                e_vec = vs[0]
                e_out[pl.ds(loc + grp * 16, 16)] = e_vec
                gmax = jnp.maximum(gmax, e_vec)
            return gmax

        n_half = n_chunks // 2
        issue_g(0, xla, xra, gsa)

        def body(i, gmax):
            l0 = 2 * i * _EK
            l1 = l0 + _EK
            wait_g(l0, xla, xra, gsa)
            w_desc(l0, xla, wsa).start()

            @pl.when(i > 0)
            def _():
                w_desc(l1, xlb, wsb).wait()
            issue_g(l1, xlb, xrb, gsb)
            gmax = compute(l0, xla, xra, gmax)

            wait_g(l1, xlb, xrb, gsb)
            w_desc(l1, xlb, wsb).start()

            @pl.when(i + 1 < n_half)
            def _():
                w_desc(l0, xla, wsa).wait()
                issue_g(l0 + 2 * _EK, xla, xra, gsa)
            gmax = compute(l1, xlb, xrb, gmax)
            return gmax

        gmax = lax.fori_loop(0, n_half, body,
                             jnp.full((16,), -jnp.inf, jnp.float32))
        w_desc(0, xla, wsa).wait()
        w_desc(0, xlb, wsb).wait()
        gmax_v[...] = gmax
        pltpu.sync_copy(e_out, e_hbm.at[pl.ds(base, per_w)])
        pltpu.sync_copy(gmax_v, wmax_hbm.at[wid])

    return k(xl, xr, src_g, dst_g, att.reshape(8, 16))


_AK = 2048


def _agg_body(dst_ref, e_ref, g_ref, gsrc_ref, num_ref, den_ref, num_acc, den_acc):
    j = pl.program_id(1)

    @pl.when(j == 0)
    def _():
        num_acc[...] = jnp.zeros_like(num_acc)
        den_acc[...] = jnp.zeros_like(den_acc)

    w = jnp.exp(e_ref[0, 0] - g_ref[0, 0])
    rows = lax.broadcasted_iota(jnp.int32, (N, _AK), 0)
    oh = jnp.where(rows == dst_ref[0, 0], w, 0.0).astype(jnp.bfloat16)
    gb = gsrc_ref[0].astype(jnp.bfloat16)
    num_acc[...] += jnp.dot(oh, gb, preferred_element_type=jnp.float32)
    den_acc[...] += jnp.dot(oh, jnp.ones((_AK, 1), jnp.bfloat16),
                            preferred_element_type=jnp.float32)

    @pl.when(j == E // _AK - 1)
    def _():
        num_ref[0] = num_acc[...]
        den_ref[0] = den_acc[...]


def _edge_aggregate(dst_l, e_all, g_arr, gsrc):
    ngs = dst_l.shape[0]
    return pl.pallas_call(
        _agg_body,
        grid=(ngs, E // _AK),
        in_specs=[
            pl.BlockSpec((1, 1, 1, _AK), lambda i, j: (i, j, 0, 0)),
            pl.BlockSpec((1, 1, 1, _AK), lambda i, j: (i, j, 0, 0)),
            pl.BlockSpec((1, 1), lambda i, j: (0, 0)),
            pl.BlockSpec((1, _AK, H), lambda i, j: (i, j, 0)),
        ],
        out_specs=[
            pl.BlockSpec((1, N, H), lambda i, j: (i, 0, 0)),
            pl.BlockSpec((1, N, 1), lambda i, j: (i, 0, 0)),
        ],
        out_shape=[
            jax.ShapeDtypeStruct((ngs, N, H), jnp.float32),
            jax.ShapeDtypeStruct((ngs, N, 1), jnp.float32),
        ],
        scratch_shapes=[
            pltpu.VMEM((N, H), jnp.float32),
            pltpu.VMEM((N, 1), jnp.float32),
        ],
    )(dst_l.reshape(ngs, E // _AK, 1, _AK),
      e_all.reshape(ngs, E // _AK, 1, _AK),
      g_arr.reshape(1, 16)[:, :1],
      gsrc.reshape(ngs, E, H))


def _proj_body(x_ref, wl_ref, bl_ref, wr_ref, br_ref, xl_ref, xr_ref):
    x = x_ref[...]
    xl_ref[...] = jnp.dot(x, wl_ref[...], preferred_element_type=jnp.float32) + bl_ref[...]
    xr_ref[...] = jnp.dot(x, wr_ref[...], preferred_element_type=jnp.float32) + br_ref[...]


def _projections(x, Wl, bl, Wr, br):
    n = x.shape[0]
    blk = 4096
    grid = (n // blk,)
    return pl.pallas_call(
        _proj_body,
        grid=grid,
        in_specs=[
            pl.BlockSpec((blk, H), lambda i: (i, 0)),
            pl.BlockSpec((H, H), lambda i: (0, 0)),
            pl.BlockSpec((1, H), lambda i: (0, 0)),
            pl.BlockSpec((H, H), lambda i: (0, 0)),
            pl.BlockSpec((1, H), lambda i: (0, 0)),
        ],
        out_specs=[
            pl.BlockSpec((blk, H), lambda i: (i, 0)),
            pl.BlockSpec((blk, H), lambda i: (i, 0)),
        ],
        out_shape=[
            jax.ShapeDtypeStruct((n, H), jnp.float32),
            jax.ShapeDtypeStruct((n, H), jnp.float32),
        ],
    )(x, Wl, bl.reshape(1, H), Wr, br.reshape(1, H))


def _pool_body(num_ref, den_ref, enc_ref, bias_ref, out_ref):
    s = pl.program_id(1)
    den = den_ref[0]
    gat = jnp.where(den > 0.0,
                    num_ref[0] / jnp.where(den > 0.0, den, 1.0), 0.0)
    gat = gat + bias_ref[...] + enc_ref[0]
    colmax = jnp.max(gat, axis=0, keepdims=True)

    @pl.when(s == 0)
    def _():
        out_ref[0] = colmax

    @pl.when(s == 1)
    def _():
        out_ref[0] = jnp.maximum(out_ref[0], colmax)


def _pool(num, den, enc, gat_bias):
    return pl.pallas_call(
        _pool_body,
        grid=(B, 2),
        in_specs=[
            pl.BlockSpec((1, N, H), lambda b, s: (s * B + b, 0, 0)),
            pl.BlockSpec((1, N, 1), lambda b, s: (s * B + b, 0, 0)),
            pl.BlockSpec((1, N, H), lambda b, s: (s * B + b, 0, 0)),
            pl.BlockSpec((1, H), lambda b, s: (0, 0)),
        ],
        out_specs=pl.BlockSpec((1, 1, H), lambda b, s: (b, 0, 0)),
        out_shape=jax.ShapeDtypeStruct((B, 1, H), jnp.float32),
    )(num, den, enc, gat_bias.reshape(1, H)).reshape(B, H)


def _mlp_body(fused_ref, W0_ref, b0_ref, W1_ref, b1_ref, W2_ref, b2_ref, out_ref):
    h = jnp.maximum(jnp.dot(fused_ref[...], W0_ref[...],
                            preferred_element_type=jnp.float32) + b0_ref[...], 0.0)
    h = jnp.maximum(jnp.dot(h, W1_ref[...],
                            preferred_element_type=jnp.float32) + b1_ref[...], 0.0)
    out_ref[...] = jnp.dot(h, W2_ref[...],
                           preferred_element_type=jnp.float32) + b2_ref[...]


def _mlp_head(fused, W0, b0, W1, b1, W2, b2):
    return pl.pallas_call(
        _mlp_body,
        out_shape=jax.ShapeDtypeStruct((B, C), jnp.float32),
    )(fused, W0, b0.reshape(1, -1), W1, b1.reshape(1, -1), W2, b2.reshape(1, -1))


def kernel(diff_input, msg_input, graph_edge_index_diff, graph_edge_index_msg,
           emb, Wl, bl, Wr, br, att, gat_bias, W0, b0, W1, b1, W2, b2):
    ids_all = jnp.concatenate(
        [diff_input[0].reshape(-1), msg_input[0].reshape(-1)]).astype(jnp.int32)
    enc_all = _gather_rows(emb, ids_all)

    xl, xr = _projections(enc_all, Wl, bl, Wr, br)

    src_l = jnp.concatenate(
        [graph_edge_index_diff[:, 0, :], graph_edge_index_msg[:, 0, :]]
    ).astype(jnp.int32)
    dst_l = jnp.concatenate(
        [graph_edge_index_diff[:, 1, :], graph_edge_index_msg[:, 1, :]]
    ).astype(jnp.int32)
    base = (jnp.arange(NGS, dtype=jnp.int32) * N)[:, None]
    src_g = (src_l + base).reshape(-1)
    dst_g = (dst_l + base).reshape(-1)

    half = NGS * E // 2
    e0, wm0, gsrc0 = _edge_logits(xl, xr, src_g[:half], dst_g[:half], att)
    e1, wm1, gsrc1 = _edge_logits(xl, xr, src_g[half:], dst_g[half:], att)
    num0, den0 = _edge_aggregate(dst_l[:NGS // 2], e0,
                                 jnp.full((16,), jnp.max(wm0), jnp.float32), gsrc0)
    num1, den1 = _edge_aggregate(dst_l[NGS // 2:], e1,
                                 jnp.full((16,), jnp.max(wm1), jnp.float32), gsrc1)
    num = jnp.concatenate([num0, num1])
    den = jnp.concatenate([den0, den1])

    enc3 = enc_all.reshape(NGS, N, H)
    fused = _pool(num, den, enc3, gat_bias)
    return _mlp_head(fused, W0, b0, W1, b1, W2, b2)


# speedup vs baseline: 1.6415x; 1.0909x over previous
import functools

import jax
import jax.numpy as jnp
from jax import lax
from jax.experimental import pallas as pl
from jax.experimental.pallas import tpu as pltpu
from jax.experimental.pallas import tpu_sc as plsc

B, N, E, H, V, C = 8, 2048, 32768, 128, 50000, 2
NGS = 2 * B

_info = plsc.get_sparse_core_info()
NC, NS = _info.num_cores, _info.num_subcores
NW = NC * NS


def _gather_rows(table, idx):
    n = idx.shape[0]
    per_w = n // NW
    chunk = 512
    mesh = plsc.VectorSubcoreMesh(core_axis_name="c", subcore_axis_name="s")

    @functools.partial(
        pl.kernel, mesh=mesh,
        out_type=jax.ShapeDtypeStruct((n, H), jnp.float32),
        scratch_types=[
            pltpu.VMEM((chunk,), jnp.int32),
            pltpu.VMEM((chunk, H), jnp.float32),
            pltpu.SemaphoreType.DMA,
        ],
    )
    def k(table_hbm, idx_hbm, out_hbm, idx_v, rows_v, sem):
        wid = lax.axis_index("s") * NC + lax.axis_index("c")
        base = wid * per_w

        def body(i, carry):
            off = base + i * chunk
            pltpu.sync_copy(idx_hbm.at[pl.ds(off, chunk)], idx_v)
            pltpu.async_copy(table_hbm.at[idx_v], rows_v, sem).wait()
            pltpu.sync_copy(rows_v, out_hbm.at[pl.ds(off, chunk)])
            return carry

        lax.fori_loop(0, per_w // chunk, body, 0)

    return k(table, idx)


_EK = 32

_GDN = lax.GatherDimensionNumbers(
    offset_dims=(), collapsed_slice_dims=(0,), start_index_map=(0,))


def _vperm(v, idx2d):
    return lax.gather(v, idx2d, _GDN, (1,),
                      mode=lax.GatherScatterMode.PROMISE_IN_BOUNDS)


def _edge_logits(xl, xr, src_g, dst_g, att):
    ne = src_g.shape[0]
    per_w = ne // NW
    n_chunks = per_w // _EK
    mesh = plsc.VectorSubcoreMesh(core_axis_name="c", subcore_axis_name="s")

    @functools.partial(
        pl.kernel, mesh=mesh,
        out_type=[
            jax.ShapeDtypeStruct((ne,), jnp.float32),
            jax.ShapeDtypeStruct((NW, 16), jnp.float32),
            jax.ShapeDtypeStruct((ne, H), jnp.float32),
        ],
        scratch_types=[
            pltpu.VMEM((per_w,), jnp.int32),
            pltpu.VMEM((per_w,), jnp.int32),
            pltpu.VMEM((_EK, H), jnp.float32),
            pltpu.VMEM((_EK, H), jnp.float32),
            pltpu.VMEM((_EK, H), jnp.float32),
            pltpu.VMEM((_EK, H), jnp.float32),
            pltpu.VMEM((8, 16), jnp.float32),
            pltpu.VMEM((per_w,), jnp.float32),
            pltpu.VMEM((16,), jnp.float32),
            pltpu.SemaphoreType.DMA,
            pltpu.SemaphoreType.DMA,
            pltpu.SemaphoreType.DMA,
            pltpu.SemaphoreType.DMA,
        ],
    )
    def k(xl_hbm, xr_hbm, src_hbm, dst_hbm, att_hbm, e_hbm, wmax_hbm, gsrc_hbm,
          src_v, dst_v, xla, xra, xlb, xrb, att_v, e_out, gmax_v,
          gsa, gsb, wsa, wsb):
        wid = lax.axis_index("s") * NC + lax.axis_index("c")
        base = wid * per_w
        pltpu.sync_copy(att_hbm, att_v)
        pltpu.sync_copy(src_hbm.at[pl.ds(base, per_w)], src_v)
        pltpu.sync_copy(dst_hbm.at[pl.ds(base, per_w)], dst_v)
        lanes = lax.iota(jnp.int32, 16)
        perms = [(lanes ^ (1 << kk)).reshape(16, 1) for kk in range(4)]

        def g_desc(loc, xlr, xrr, sem):
            return (pltpu.make_async_copy(
                        xl_hbm.at[src_v.at[pl.ds(loc, _EK)]], xlr, sem),
                    pltpu.make_async_copy(
                        xr_hbm.at[dst_v.at[pl.ds(loc, _EK)]], xrr, sem))

        def w_desc(loc, xlr, sem):
            return pltpu.make_async_copy(xlr, gsrc_hbm.at[pl.ds(base + loc, _EK)], sem)

        def issue_g(loc, xlr, xrr, sem):
            a, b = g_desc(loc, xlr, xrr, sem)
            a.start()
            b.start()

        def wait_g(loc, xlr, xrr, sem):
            a, b = g_desc(loc, xlr, xrr, sem)
            a.wait()
            b.wait()

        def compute(loc, xlr, xrr, gmax):
            for grp in range(_EK // 16):
                vs = []
                for jj in range(16):
                    j = grp * 16 + jj
                    acc = jnp.zeros((16,), jnp.float32)
                    for h8 in range(8):
                        m = xlr[j, pl.ds(16 * h8, 16)] + xrr[j, pl.ds(16 * h8, 16)]
                        lr = jnp.maximum(m, 0.2 * m)
                        acc = acc + lr * att_v[h8]
                    vs.append(acc)
                for kk in range(4):
                    sel = ((lanes >> kk) & 1) == 0
                    perm = perms[kk]
                    vs = [jnp.where(sel,
                                    vs[2 * t] + _vperm(vs[2 * t], perm),
                                    vs[2 * t + 1] + _vperm(vs[2 * t + 1], perm))
                          for t in range(len(vs) // 2)]
                e_vec = vs[0]
                e_out[pl.ds(loc + grp * 16, 16)] = e_vec
                gmax = jnp.maximum(gmax, e_vec)
            return gmax

        n_half = n_chunks // 2
        issue_g(0, xla, xra, gsa)

        def body(i, gmax):
            l0 = 2 * i * _EK
            l1 = l0 + _EK
            wait_g(l0, xla, xra, gsa)
            w_desc(l0, xla, wsa).start()

            @pl.when(i > 0)
            def _():
                w_desc(l1, xlb, wsb).wait()
            issue_g(l1, xlb, xrb, gsb)
            gmax = compute(l0, xla, xra, gmax)

            wait_g(l1, xlb, xrb, gsb)
            w_desc(l1, xlb, wsb).start()

            @pl.when(i + 1 < n_half)
            def _():
                w_desc(l0, xla, wsa).wait()
                issue_g(l0 + 2 * _EK, xla, xra, gsa)
            gmax = compute(l1, xlb, xrb, gmax)
            return gmax

        gmax = lax.fori_loop(0, n_half, body,
                             jnp.full((16,), -jnp.inf, jnp.float32))
        w_desc(0, xla, wsa).wait()
        w_desc(0, xlb, wsb).wait()
        gmax_v[...] = gmax
        pltpu.sync_copy(e_out, e_hbm.at[pl.ds(base, per_w)])
        pltpu.sync_copy(gmax_v, wmax_hbm.at[wid])

    return k(xl, xr, src_g, dst_g, att.reshape(8, 16))


_AK = 2048


def _agg_body(dst_ref, e_ref, g_ref, gsrc_ref, num_ref, den_ref, num_acc, den_acc):
    j = pl.program_id(1)

    @pl.when(j == 0)
    def _():
        num_acc[...] = jnp.zeros_like(num_acc)
        den_acc[...] = jnp.zeros_like(den_acc)

    w = jnp.exp(e_ref[0, 0] - g_ref[0, 0])
    rows = lax.broadcasted_iota(jnp.int32, (N, _AK), 0)
    oh = jnp.where(rows == dst_ref[0, 0], w, 0.0).astype(jnp.bfloat16)
    gb = gsrc_ref[0].astype(jnp.bfloat16)
    num_acc[...] += jnp.dot(oh, gb, preferred_element_type=jnp.float32)
    den_acc[...] += jnp.dot(oh, jnp.ones((_AK, 1), jnp.bfloat16),
                            preferred_element_type=jnp.float32)

    @pl.when(j == E // _AK - 1)
    def _():
        num_ref[0] = num_acc[...]
        den_ref[0] = den_acc[...]


def _edge_aggregate(dst_l, e_all, g_arr, gsrc):
    ngs = dst_l.shape[0]
    return pl.pallas_call(
        _agg_body,
        grid=(ngs, E // _AK),
        in_specs=[
            pl.BlockSpec((1, 1, 1, _AK), lambda i, j: (i, j, 0, 0)),
            pl.BlockSpec((1, 1, 1, _AK), lambda i, j: (i, j, 0, 0)),
            pl.BlockSpec((1, 1), lambda i, j: (0, 0)),
            pl.BlockSpec((1, _AK, H), lambda i, j: (i, j, 0)),
        ],
        out_specs=[
            pl.BlockSpec((1, N, H), lambda i, j: (i, 0, 0)),
            pl.BlockSpec((1, N, 1), lambda i, j: (i, 0, 0)),
        ],
        out_shape=[
            jax.ShapeDtypeStruct((ngs, N, H), jnp.float32),
            jax.ShapeDtypeStruct((ngs, N, 1), jnp.float32),
        ],
        scratch_shapes=[
            pltpu.VMEM((N, H), jnp.float32),
            pltpu.VMEM((N, 1), jnp.float32),
        ],
    )(dst_l.reshape(ngs, E // _AK, 1, _AK),
      e_all.reshape(ngs, E // _AK, 1, _AK),
      g_arr.reshape(1, 16)[:, :1],
      gsrc.reshape(ngs, E, H))


def _proj_body(x_ref, wl_ref, bl_ref, wr_ref, br_ref, xl_ref, xr_ref):
    x = x_ref[...]
    xl_ref[...] = jnp.dot(x, wl_ref[...], preferred_element_type=jnp.float32) + bl_ref[...]
    xr_ref[...] = jnp.dot(x, wr_ref[...], preferred_element_type=jnp.float32) + br_ref[...]


def _projections(x, Wl, bl, Wr, br):
    n = x.shape[0]
    blk = 4096
    grid = (n // blk,)
    return pl.pallas_call(
        _proj_body,
        grid=grid,
        in_specs=[
            pl.BlockSpec((blk, H), lambda i: (i, 0)),
            pl.BlockSpec((H, H), lambda i: (0, 0)),
            pl.BlockSpec((1, H), lambda i: (0, 0)),
            pl.BlockSpec((H, H), lambda i: (0, 0)),
            pl.BlockSpec((1, H), lambda i: (0, 0)),
        ],
        out_specs=[
            pl.BlockSpec((blk, H), lambda i: (i, 0)),
            pl.BlockSpec((blk, H), lambda i: (i, 0)),
        ],
        out_shape=[
            jax.ShapeDtypeStruct((n, H), jnp.float32),
            jax.ShapeDtypeStruct((n, H), jnp.float32),
        ],
    )(x, Wl, bl.reshape(1, H), Wr, br.reshape(1, H))


def _pool_body(num_ref, den_ref, enc_ref, bias_ref, out_ref):
    s = pl.program_id(1)
    den = den_ref[0]
    gat = jnp.where(den > 0.0,
                    num_ref[0] / jnp.where(den > 0.0, den, 1.0), 0.0)
    gat = gat + bias_ref[...] + enc_ref[0]
    colmax = jnp.max(gat, axis=0, keepdims=True)

    @pl.when(s == 0)
    def _():
        out_ref[0] = colmax

    @pl.when(s == 1)
    def _():
        out_ref[0] = jnp.maximum(out_ref[0], colmax)


def _pool(num, den, enc, gat_bias):
    return pl.pallas_call(
        _pool_body,
        grid=(B, 2),
        in_specs=[
            pl.BlockSpec((1, N, H), lambda b, s: (s * B + b, 0, 0)),
            pl.BlockSpec((1, N, 1), lambda b, s: (s * B + b, 0, 0)),
            pl.BlockSpec((1, N, H), lambda b, s: (s * B + b, 0, 0)),
            pl.BlockSpec((1, H), lambda b, s: (0, 0)),
        ],
        out_specs=pl.BlockSpec((1, 1, H), lambda b, s: (b, 0, 0)),
        out_shape=jax.ShapeDtypeStruct((B, 1, H), jnp.float32),
    )(num, den, enc, gat_bias.reshape(1, H)).reshape(B, H)


def _mlp_body(fused_ref, W0_ref, b0_ref, W1_ref, b1_ref, W2_ref, b2_ref, out_ref):
    h = jnp.maximum(jnp.dot(fused_ref[...], W0_ref[...],
                            preferred_element_type=jnp.float32) + b0_ref[...], 0.0)
    h = jnp.maximum(jnp.dot(h, W1_ref[...],
                            preferred_element_type=jnp.float32) + b1_ref[...], 0.0)
    out_ref[...] = jnp.dot(h, W2_ref[...],
                           preferred_element_type=jnp.float32) + b2_ref[...]


def _mlp_head(fused, W0, b0, W1, b1, W2, b2):
    return pl.pallas_call(
        _mlp_body,
        out_shape=jax.ShapeDtypeStruct((B, C), jnp.float32),
    )(fused, W0, b0.reshape(1, -1), W1, b1.reshape(1, -1), W2, b2.reshape(1, -1))


def kernel(diff_input, msg_input, graph_edge_index_diff, graph_edge_index_msg,
           emb, Wl, bl, Wr, br, att, gat_bias, W0, b0, W1, b1, W2, b2):
    ids_all = jnp.concatenate(
        [diff_input[0].reshape(-1), msg_input[0].reshape(-1)]).astype(jnp.int32)
    enc_all = _gather_rows(emb, ids_all)

    xl, xr = _projections(enc_all, Wl, bl, Wr, br)

    src_l = jnp.concatenate(
        [graph_edge_index_diff[:, 0, :], graph_edge_index_msg[:, 0, :]]
    ).astype(jnp.int32)
    dst_l = jnp.concatenate(
        [graph_edge_index_diff[:, 1, :], graph_edge_index_msg[:, 1, :]]
    ).astype(jnp.int32)
    base = (jnp.arange(NGS, dtype=jnp.int32) * N)[:, None]
    src_g = (src_l + base).reshape(-1)
    dst_g = (dst_l + base).reshape(-1)

    nsplit = 4
    part = NGS * E // nsplit
    gpart = NGS // nsplit
    logits = [_edge_logits(xl, xr, src_g[p * part:(p + 1) * part],
                           dst_g[p * part:(p + 1) * part], att)
              for p in range(nsplit)]
    aggs = [_edge_aggregate(dst_l[p * gpart:(p + 1) * gpart], logits[p][0],
                            jnp.full((16,), jnp.max(logits[p][1]), jnp.float32),
                            logits[p][2])
            for p in range(nsplit)]
    num = jnp.concatenate([a[0] for a in aggs])
    den = jnp.concatenate([a[1] for a in aggs])

    enc3 = enc_all.reshape(NGS, N, H)
    fused = _pool(num, den, enc3, gat_bias)
    return _mlp_head(fused, W0, b0, W1, b1, W2, b2)


# speedup vs baseline: 1.6878x; 1.0283x over previous
import functools

import jax
import jax.numpy as jnp
from jax import lax
from jax.experimental import pallas as pl
from jax.experimental.pallas import tpu as pltpu
from jax.experimental.pallas import tpu_sc as plsc

B, N, E, H, V, C = 8, 2048, 32768, 128, 50000, 2
NGS = 2 * B

_info = plsc.get_sparse_core_info()
NC, NS = _info.num_cores, _info.num_subcores
NW = NC * NS


def _gather_rows(table, idx):
    n = idx.shape[0]
    per_w = n // NW
    chunk = 512
    mesh = plsc.VectorSubcoreMesh(core_axis_name="c", subcore_axis_name="s")

    @functools.partial(
        pl.kernel, mesh=mesh,
        out_type=jax.ShapeDtypeStruct((n, H), jnp.float32),
        scratch_types=[
            pltpu.VMEM((chunk,), jnp.int32),
            pltpu.VMEM((chunk, H), jnp.float32),
            pltpu.SemaphoreType.DMA,
        ],
    )
    def k(table_hbm, idx_hbm, out_hbm, idx_v, rows_v, sem):
        wid = lax.axis_index("s") * NC + lax.axis_index("c")
        base = wid * per_w

        def body(i, carry):
            off = base + i * chunk
            pltpu.sync_copy(idx_hbm.at[pl.ds(off, chunk)], idx_v)
            pltpu.async_copy(table_hbm.at[idx_v], rows_v, sem).wait()
            pltpu.sync_copy(rows_v, out_hbm.at[pl.ds(off, chunk)])
            return carry

        lax.fori_loop(0, per_w // chunk, body, 0)

    return k(table, idx)


_EK = 32

_GDN = lax.GatherDimensionNumbers(
    offset_dims=(), collapsed_slice_dims=(0,), start_index_map=(0,))


def _vperm(v, idx2d):
    return lax.gather(v, idx2d, _GDN, (1,),
                      mode=lax.GatherScatterMode.PROMISE_IN_BOUNDS)


def _edge_logits(xl, xr, src_g, dst_g, att):
    ne = src_g.shape[0]
    per_w = ne // NW
    n_chunks = per_w // _EK
    mesh = plsc.VectorSubcoreMesh(core_axis_name="c", subcore_axis_name="s")

    @functools.partial(
        pl.kernel, mesh=mesh,
        out_type=[
            jax.ShapeDtypeStruct((ne,), jnp.float32),
            jax.ShapeDtypeStruct((NW, 16), jnp.float32),
            jax.ShapeDtypeStruct((ne, H), jnp.float32),
        ],
        scratch_types=[
            pltpu.VMEM((per_w,), jnp.int32),
            pltpu.VMEM((per_w,), jnp.int32),
            pltpu.VMEM((_EK, H), jnp.float32),
            pltpu.VMEM((_EK, H), jnp.float32),
            pltpu.VMEM((_EK, H), jnp.float32),
            pltpu.VMEM((_EK, H), jnp.float32),
            pltpu.VMEM((8, 16), jnp.float32),
            pltpu.VMEM((per_w,), jnp.float32),
            pltpu.VMEM((16,), jnp.float32),
            pltpu.SemaphoreType.DMA,
            pltpu.SemaphoreType.DMA,
            pltpu.SemaphoreType.DMA,
            pltpu.SemaphoreType.DMA,
        ],
    )
    def k(xl_hbm, xr_hbm, src_hbm, dst_hbm, att_hbm, e_hbm, wmax_hbm, gsrc_hbm,
          src_v, dst_v, xla, xra, xlb, xrb, att_v, e_out, gmax_v,
          gsa, gsb, wsa, wsb):
        wid = lax.axis_index("s") * NC + lax.axis_index("c")
        base = wid * per_w
        pltpu.sync_copy(att_hbm, att_v)
        pltpu.sync_copy(src_hbm.at[pl.ds(base, per_w)], src_v)
        pltpu.sync_copy(dst_hbm.at[pl.ds(base, per_w)], dst_v)
        lanes = lax.iota(jnp.int32, 16)
        perms = [(lanes ^ (1 << kk)).reshape(16, 1) for kk in range(4)]

        def g_desc(loc, xlr, xrr, sem):
            return (pltpu.make_async_copy(
                        xl_hbm.at[src_v.at[pl.ds(loc, _EK)]], xlr, sem),
                    pltpu.make_async_copy(
                        xr_hbm.at[dst_v.at[pl.ds(loc, _EK)]], xrr, sem))

        def w_desc(loc, xlr, sem):
            return pltpu.make_async_copy(xlr, gsrc_hbm.at[pl.ds(base + loc, _EK)], sem)

        def issue_g(loc, xlr, xrr, sem):
            a, b = g_desc(loc, xlr, xrr, sem)
            a.start()
            b.start()

        def wait_g(loc, xlr, xrr, sem):
            a, b = g_desc(loc, xlr, xrr, sem)
            a.wait()
            b.wait()

        def compute(loc, xlr, xrr, gmax):
            for grp in range(_EK // 16):
                vs = []
                for jj in range(16):
                    j = grp * 16 + jj
                    acc = jnp.zeros((16,), jnp.float32)
                    for h8 in range(8):
                        m = xlr[j, pl.ds(16 * h8, 16)] + xrr[j, pl.ds(16 * h8, 16)]
                        lr = jnp.maximum(m, 0.2 * m)
                        acc = acc + lr * att_v[h8]
                    vs.append(acc)
                for kk in range(4):
                    sel = ((lanes >> kk) & 1) == 0
                    perm = perms[kk]
                    vs = [jnp.where(sel,
                                    vs[2 * t] + _vperm(vs[2 * t], perm),
                                    vs[2 * t + 1] + _vperm(vs[2 * t + 1], perm))
                          for t in range(len(vs) // 2)]
                e_vec = vs[0]
                e_out[pl.ds(loc + grp * 16, 16)] = e_vec
                gmax = jnp.maximum(gmax, e_vec)
            return gmax

        n_half = n_chunks // 2
        issue_g(0, xla, xra, gsa)

        def body(i, gmax):
            l0 = 2 * i * _EK
            l1 = l0 + _EK
            wait_g(l0, xla, xra, gsa)
            w_desc(l0, xla, wsa).start()

            @pl.when(i > 0)
            def _():
                w_desc(l1, xlb, wsb).wait()
            issue_g(l1, xlb, xrb, gsb)
            gmax = compute(l0, xla, xra, gmax)

            wait_g(l1, xlb, xrb, gsb)
            w_desc(l1, xlb, wsb).start()

            @pl.when(i + 1 < n_half)
            def _():
                w_desc(l0, xla, wsa).wait()
                issue_g(l0 + 2 * _EK, xla, xra, gsa)
            gmax = compute(l1, xlb, xrb, gmax)
            return gmax

        gmax = lax.fori_loop(0, n_half, body,
                             jnp.full((16,), -jnp.inf, jnp.float32))
        w_desc(0, xla, wsa).wait()
        w_desc(0, xlb, wsb).wait()
        gmax_v[...] = gmax
        pltpu.sync_copy(e_out, e_hbm.at[pl.ds(base, per_w)])
        pltpu.sync_copy(gmax_v, wmax_hbm.at[wid])

    return k(xl, xr, src_g, dst_g, att.reshape(8, 16))


_AK = 2048


def _agg_body(dst_ref, e_ref, g_ref, gsrc_ref, num_ref, den_ref, num_acc, den_acc):
    j = pl.program_id(1)

    @pl.when(j == 0)
    def _():
        num_acc[...] = jnp.zeros_like(num_acc)
        den_acc[...] = jnp.zeros_like(den_acc)

    w = jnp.exp(e_ref[0, 0] - g_ref[0, 0])
    rows = lax.broadcasted_iota(jnp.int32, (N, _AK), 0)
    oh = jnp.where(rows == dst_ref[0, 0], w, 0.0).astype(jnp.bfloat16)
    gb = gsrc_ref[0].astype(jnp.bfloat16)
    num_acc[...] += jnp.dot(oh, gb, preferred_element_type=jnp.float32)
    den_acc[...] += jnp.dot(oh, jnp.ones((_AK, 1), jnp.bfloat16),
                            preferred_element_type=jnp.float32)

    @pl.when(j == E // _AK - 1)
    def _():
        num_ref[0] = num_acc[...]
        den_ref[0] = den_acc[...]


def _edge_aggregate(dst_l, e_all, g_arr, gsrc):
    ngs = dst_l.shape[0]
    return pl.pallas_call(
        _agg_body,
        grid=(ngs, E // _AK),
        in_specs=[
            pl.BlockSpec((1, 1, 1, _AK), lambda i, j: (i, j, 0, 0)),
            pl.BlockSpec((1, 1, 1, _AK), lambda i, j: (i, j, 0, 0)),
            pl.BlockSpec((1, 1), lambda i, j: (0, 0)),
            pl.BlockSpec((1, _AK, H), lambda i, j: (i, j, 0)),
        ],
        out_specs=[
            pl.BlockSpec((1, N, H), lambda i, j: (i, 0, 0)),
            pl.BlockSpec((1, N, 1), lambda i, j: (i, 0, 0)),
        ],
        out_shape=[
            jax.ShapeDtypeStruct((ngs, N, H), jnp.float32),
            jax.ShapeDtypeStruct((ngs, N, 1), jnp.float32),
        ],
        scratch_shapes=[
            pltpu.VMEM((N, H), jnp.float32),
            pltpu.VMEM((N, 1), jnp.float32),
        ],
    )(dst_l.reshape(ngs, E // _AK, 1, _AK),
      e_all.reshape(ngs, E // _AK, 1, _AK),
      g_arr.reshape(1, 16)[:, :1],
      gsrc.reshape(ngs, E, H))


def _proj_body(x_ref, wl_ref, bl_ref, wr_ref, br_ref, xl_ref, xr_ref):
    x = x_ref[...]
    xl_ref[...] = jnp.dot(x, wl_ref[...], preferred_element_type=jnp.float32) + bl_ref[...]
    xr_ref[...] = jnp.dot(x, wr_ref[...], preferred_element_type=jnp.float32) + br_ref[...]


def _projections(x, Wl, bl, Wr, br):
    n = x.shape[0]
    blk = 4096
    grid = (n // blk,)
    return pl.pallas_call(
        _proj_body,
        grid=grid,
        in_specs=[
            pl.BlockSpec((blk, H), lambda i: (i, 0)),
            pl.BlockSpec((H, H), lambda i: (0, 0)),
            pl.BlockSpec((1, H), lambda i: (0, 0)),
            pl.BlockSpec((H, H), lambda i: (0, 0)),
            pl.BlockSpec((1, H), lambda i: (0, 0)),
        ],
        out_specs=[
            pl.BlockSpec((blk, H), lambda i: (i, 0)),
            pl.BlockSpec((blk, H), lambda i: (i, 0)),
        ],
        out_shape=[
            jax.ShapeDtypeStruct((n, H), jnp.float32),
            jax.ShapeDtypeStruct((n, H), jnp.float32),
        ],
    )(x, Wl, bl.reshape(1, H), Wr, br.reshape(1, H))


def _pool_body(num_ref, den_ref, enc_ref, bias_ref, out_ref):
    s = pl.program_id(1)
    den = den_ref[0]
    gat = jnp.where(den > 0.0,
                    num_ref[0] / jnp.where(den > 0.0, den, 1.0), 0.0)
    gat = gat + bias_ref[...] + enc_ref[0]
    colmax = jnp.max(gat, axis=0, keepdims=True)

    @pl.when(s == 0)
    def _():
        out_ref[0] = colmax

    @pl.when(s == 1)
    def _():
        out_ref[0] = jnp.maximum(out_ref[0], colmax)


def _pool(num, den, enc, gat_bias):
    return pl.pallas_call(
        _pool_body,
        grid=(B, 2),
        in_specs=[
            pl.BlockSpec((1, N, H), lambda b, s: (s * B + b, 0, 0)),
            pl.BlockSpec((1, N, 1), lambda b, s: (s * B + b, 0, 0)),
            pl.BlockSpec((1, N, H), lambda b, s: (s * B + b, 0, 0)),
            pl.BlockSpec((1, H), lambda b, s: (0, 0)),
        ],
        out_specs=pl.BlockSpec((1, 1, H), lambda b, s: (b, 0, 0)),
        out_shape=jax.ShapeDtypeStruct((B, 1, H), jnp.float32),
    )(num, den, enc, gat_bias.reshape(1, H)).reshape(B, H)


def _mlp_body(fused_ref, W0_ref, b0_ref, W1_ref, b1_ref, W2_ref, b2_ref, out_ref):
    h = jnp.maximum(jnp.dot(fused_ref[...], W0_ref[...],
                            preferred_element_type=jnp.float32) + b0_ref[...], 0.0)
    h = jnp.maximum(jnp.dot(h, W1_ref[...],
                            preferred_element_type=jnp.float32) + b1_ref[...], 0.0)
    out_ref[...] = jnp.dot(h, W2_ref[...],
                           preferred_element_type=jnp.float32) + b2_ref[...]


def _mlp_head(fused, W0, b0, W1, b1, W2, b2):
    return pl.pallas_call(
        _mlp_body,
        out_shape=jax.ShapeDtypeStruct((B, C), jnp.float32),
    )(fused, W0, b0.reshape(1, -1), W1, b1.reshape(1, -1), W2, b2.reshape(1, -1))


def kernel(diff_input, msg_input, graph_edge_index_diff, graph_edge_index_msg,
           emb, Wl, bl, Wr, br, att, gat_bias, W0, b0, W1, b1, W2, b2):
    ids_all = jnp.concatenate(
        [diff_input[0].reshape(-1), msg_input[0].reshape(-1)]).astype(jnp.int32)
    enc_all = _gather_rows(emb, ids_all)

    xl, xr = _projections(enc_all, Wl, bl, Wr, br)

    src_l = jnp.concatenate(
        [graph_edge_index_diff[:, 0, :], graph_edge_index_msg[:, 0, :]]
    ).astype(jnp.int32)
    dst_l = jnp.concatenate(
        [graph_edge_index_diff[:, 1, :], graph_edge_index_msg[:, 1, :]]
    ).astype(jnp.int32)
    base = (jnp.arange(NGS, dtype=jnp.int32) * N)[:, None]
    src_g = (src_l + base).reshape(-1)
    dst_g = (dst_l + base).reshape(-1)

    nsplit = 8
    part = NGS * E // nsplit
    gpart = NGS // nsplit
    logits = [_edge_logits(xl, xr, src_g[p * part:(p + 1) * part],
                           dst_g[p * part:(p + 1) * part], att)
              for p in range(nsplit)]
    aggs = [_edge_aggregate(dst_l[p * gpart:(p + 1) * gpart], logits[p][0],
                            jnp.full((16,), jnp.max(logits[p][1]), jnp.float32),
                            logits[p][2])
            for p in range(nsplit)]
    num = jnp.concatenate([a[0] for a in aggs])
    den = jnp.concatenate([a[1] for a in aggs])

    enc3 = enc_all.reshape(NGS, N, H)
    fused = _pool(num, den, enc3, gat_bias)
    return _mlp_head(fused, W0, b0, W1, b1, W2, b2)
